# baseline (speedup 1.0000x reference)
import jax
import jax.numpy as jnp
from jax import lax
from jax.experimental import pallas as pl
from jax.experimental.pallas import tpu as pltpu

_TILE = 512


def _chamfer_body(px_ref, gxt_ref, out_ref):
    gxt = gxt_ref[0]
    n1 = px_ref.shape[1]
    n2 = gxt.shape[1]
    y2 = jnp.sum(gxt * gxt, axis=0, keepdims=True)

    def body(i, carry):
        sum_x, min_y = carry
        px_t = px_ref[0, pl.ds(i * _TILE, _TILE), :]
        xy = lax.dot_general(px_t, gxt, (((1,), (0,)), ((), ())),
                             preferred_element_type=jnp.float32)
        x2 = jnp.sum(px_t * px_t, axis=1, keepdims=True)
        d = jnp.maximum(x2 + y2 - 2.0 * xy, 0.0)
        sum_x = sum_x + jnp.sum(jnp.min(d, axis=1, keepdims=True),
                                axis=(0, 1), keepdims=True)
        min_y = jnp.minimum(min_y, jnp.min(d, axis=0, keepdims=True))
        return sum_x, min_y

    sum_x, min_y = lax.fori_loop(
        0, n1 // _TILE, body,
        (jnp.zeros((1, 1), dtype=jnp.float32),
         jnp.full((1, n2), jnp.inf, dtype=jnp.float32)))
    out_ref[0, :, :] = (sum_x / n1
                        + jnp.sum(min_y, axis=(0, 1), keepdims=True) / n2)


def kernel(pred_points, gt_points):
    B, N, D = pred_points.shape
    gt_t = jnp.swapaxes(gt_points, 1, 2)
    per_batch = pl.pallas_call(
        _chamfer_body,
        grid=(B,),
        in_specs=[
            pl.BlockSpec((1, N, D), lambda b: (b, 0, 0)),
            pl.BlockSpec((1, D, gt_t.shape[2]), lambda b: (b, 0, 0)),
        ],
        out_specs=pl.BlockSpec((1, 1, 1), lambda b: (b, 0, 0)),
        out_shape=jax.ShapeDtypeStruct((B, 1, 1), jnp.float32),
        compiler_params=pltpu.CompilerParams(
            dimension_semantics=("parallel",)),
    )(pred_points, gt_t)
    return jnp.mean(per_batch)


# speedup vs baseline: 1.0783x; 1.0783x over previous
import jax
import jax.numpy as jnp
from jax import lax
from jax.experimental import pallas as pl
from jax.experimental.pallas import tpu as pltpu

_TILE = 512


def _chamfer_body(px_ref, gxt_ref, out_ref):
    gxt = gxt_ref[0]
    n1 = px_ref.shape[1]
    n2 = gxt.shape[1]
    y2 = jnp.sum(gxt * gxt, axis=0, keepdims=True)

    def body(i, carry):
        sum_x, min_f = carry
        px_t = px_ref[0, pl.ds(i * _TILE, _TILE), :]
        xy = lax.dot_general(px_t, gxt, (((1,), (0,)), ((), ())),
                             preferred_element_type=jnp.float32)
        x2 = jnp.sum(px_t * px_t, axis=1, keepdims=True)
        e = y2 - 2.0 * xy
        f = x2 - 2.0 * xy
        cham_x_t = jnp.maximum(jnp.min(e, axis=1, keepdims=True) + x2, 0.0)
        sum_x = sum_x + jnp.sum(cham_x_t, axis=(0, 1), keepdims=True)
        min_f = jnp.minimum(min_f, jnp.min(f, axis=0, keepdims=True))
        return sum_x, min_f

    sum_x, min_f = lax.fori_loop(
        0, n1 // _TILE, body,
        (jnp.zeros((1, 1), dtype=jnp.float32),
         jnp.full((1, n2), jnp.inf, dtype=jnp.float32)))
    cham_y = jnp.maximum(min_f + y2, 0.0)
    out_ref[0, :, :] = (sum_x / n1
                        + jnp.sum(cham_y, axis=(0, 1), keepdims=True) / n2)


def kernel(pred_points, gt_points):
    B, N, D = pred_points.shape
    gt_t = jnp.swapaxes(gt_points, 1, 2)
    per_batch = pl.pallas_call(
        _chamfer_body,
        grid=(B,),
        in_specs=[
            pl.BlockSpec((1, N, D), lambda b: (b, 0, 0)),
            pl.BlockSpec((1, D, gt_t.shape[2]), lambda b: (b, 0, 0)),
        ],
        out_specs=pl.BlockSpec((1, 1, 1), lambda b: (b, 0, 0)),
        out_shape=jax.ShapeDtypeStruct((B, 1, 1), jnp.float32),
        compiler_params=pltpu.CompilerParams(
            dimension_semantics=("parallel",)),
    )(pred_points, gt_t)
    return jnp.mean(per_batch)


# speedup vs baseline: 1.1551x; 1.0712x over previous
import jax
import jax.numpy as jnp
from jax import lax
from jax.experimental import pallas as pl
from jax.experimental.pallas import tpu as pltpu

_TILE = 512


def _chamfer_body(px_ref, gxt_ref, out_ref):
    gxt2 = gxt_ref[0]
    n1 = px_ref.shape[1]
    n2 = gxt2.shape[1]
    y2 = 0.25 * jnp.sum(gxt2 * gxt2, axis=0, keepdims=True)

    def body(i, carry):
        sum_x, min_f = carry
        px_t = px_ref[0, pl.ds(i * _TILE, _TILE), :]
        xy2 = lax.dot_general(px_t, gxt2, (((1,), (0,)), ((), ())),
                              preferred_element_type=jnp.float32)
        x2 = jnp.sum(px_t * px_t, axis=1, keepdims=True)
        e = y2 + xy2
        f = x2 + xy2
        cham_x_t = jnp.maximum(jnp.min(e, axis=1, keepdims=True) + x2, 0.0)
        sum_x = sum_x + jnp.sum(cham_x_t, axis=(0, 1), keepdims=True)
        min_f = jnp.minimum(min_f, jnp.min(f, axis=0, keepdims=True))
        return sum_x, min_f

    sum_x, min_f = lax.fori_loop(
        0, n1 // _TILE, body,
        (jnp.zeros((1, 1), dtype=jnp.float32),
         jnp.full((1, n2), jnp.inf, dtype=jnp.float32)))
    cham_y = jnp.maximum(min_f + y2, 0.0)
    out_ref[0, :, :] = (sum_x / n1
                        + jnp.sum(cham_y, axis=(0, 1), keepdims=True) / n2)


def kernel(pred_points, gt_points):
    B, N, D = pred_points.shape
    gt_t = jnp.swapaxes(gt_points, 1, 2) * jnp.float32(-2.0)
    per_batch = pl.pallas_call(
        _chamfer_body,
        grid=(B,),
        in_specs=[
            pl.BlockSpec((1, N, D), lambda b: (b, 0, 0)),
            pl.BlockSpec((1, D, gt_t.shape[2]), lambda b: (b, 0, 0)),
        ],
        out_specs=pl.BlockSpec((1, 1, 1), lambda b: (b, 0, 0)),
        out_shape=jax.ShapeDtypeStruct((B, 1, 1), jnp.float32),
        compiler_params=pltpu.CompilerParams(
            dimension_semantics=("parallel",)),
    )(pred_points, gt_t)
    return jnp.mean(per_batch)
